# Initial kernel scaffold; baseline (speedup 1.0000x reference)
import jax
import jax.numpy as jnp
from jax.experimental import pallas as pl


def kernel(x, adj, Wv, bv, wq, bq, wk, bk):
    raise NotImplementedError("write your pallas kernel here")


# speedup vs baseline: 1.2114x; 1.2114x over previous
import functools

import jax
import jax.numpy as jnp
from jax.experimental import pallas as pl
from jax.experimental.pallas import tpu as pltpu


def _proj_kernel(x_ref, wv_ref, bv_ref, wq_ref, bq_ref, wk_ref, bk_ref,
                 xv_ref, el_ref, er_ref, elmax_ref):
    i = pl.program_id(0)
    xv = jax.lax.dot_general(
        x_ref[...], wv_ref[...], (((1,), (1,)), ((), ())),
        preferred_element_type=jnp.float32) + bv_ref[...]
    xv_ref[...] = xv
    el = jnp.sum(xv * wq_ref[...], axis=1, keepdims=True) + bq_ref[0, 0]
    el_ref[...] = el
    er_col = jnp.sum(xv * wk_ref[...], axis=1, keepdims=True) + bk_ref[0, 0]
    er_ref[...] = er_col.T
    bmax = jnp.max(el, keepdims=True)

    @pl.when(i == 0)
    def _():
        elmax_ref[...] = bmax

    @pl.when(i > 0)
    def _():
        elmax_ref[...] = jnp.maximum(elmax_ref[...], bmax)


def _attn_kernel(adj_ref, el_ref, er_ref, xv_ref, elmax_ref, out_ref,
                 acc_ref, d_ref, *, ni):
    i = pl.program_id(1)

    @pl.when(i == 0)
    def _():
        acc_ref[...] = jnp.zeros_like(acc_ref)
        d_ref[...] = jnp.zeros_like(d_ref)

    el = el_ref[...]
    er = er_ref[...]
    e = el + er
    coeff = jnp.where(e > 0, e, 0.2 * e)
    mtop = elmax_ref[...] + er
    mj = jnp.where(mtop > 0, mtop, 0.2 * mtop)
    mask = adj_ref[...] > 0
    p = jnp.where(mask, jnp.exp(coeff - mj), 0.0)
    d_ref[...] += jnp.sum(p, axis=0, keepdims=True)
    acc_ref[...] += jax.lax.dot_general(
        p, xv_ref[...], (((0,), (0,)), ((), ())),
        preferred_element_type=jnp.float32)

    @pl.when(i == ni - 1)
    def _():
        d = jnp.maximum(d_ref[...], 1e-20)
        out_ref[...] = acc_ref[...] * (1.0 / d).T


def kernel(x, adj, Wv, bv, wq, bq, wk, bk):
    n, _ = x.shape
    f = Wv.shape[0]

    bi1 = min(512, n)
    ni1 = n // bi1
    xv, el, er, elmax = pl.pallas_call(
        _proj_kernel,
        grid=(ni1,),
        in_specs=[
            pl.BlockSpec((bi1, x.shape[1]), lambda i: (i, 0)),
            pl.BlockSpec(Wv.shape, lambda i: (0, 0)),
            pl.BlockSpec((1, f), lambda i: (0, 0)),
            pl.BlockSpec((1, f), lambda i: (0, 0)),
            pl.BlockSpec((1, 1), lambda i: (0, 0)),
            pl.BlockSpec((1, f), lambda i: (0, 0)),
            pl.BlockSpec((1, 1), lambda i: (0, 0)),
        ],
        out_specs=[
            pl.BlockSpec((bi1, f), lambda i: (i, 0)),
            pl.BlockSpec((bi1, 1), lambda i: (i, 0)),
            pl.BlockSpec((1, bi1), lambda i: (0, i)),
            pl.BlockSpec((1, 1), lambda i: (0, 0)),
        ],
        out_shape=[
            jax.ShapeDtypeStruct((n, f), jnp.float32),
            jax.ShapeDtypeStruct((n, 1), jnp.float32),
            jax.ShapeDtypeStruct((1, n), jnp.float32),
            jax.ShapeDtypeStruct((1, 1), jnp.float32),
        ],
        compiler_params=pltpu.CompilerParams(
            dimension_semantics=("arbitrary",)),
    )(x, Wv, bv.reshape(1, f), wq, bq.reshape(1, 1), wk, bk.reshape(1, 1))

    bi = min(512, n)
    bj = min(512, n)
    ni = n // bi
    nj = n // bj
    out = pl.pallas_call(
        functools.partial(_attn_kernel, ni=ni),
        grid=(nj, ni),
        in_specs=[
            pl.BlockSpec((bi, bj), lambda j, i: (i, j)),
            pl.BlockSpec((bi, 1), lambda j, i: (i, 0)),
            pl.BlockSpec((1, bj), lambda j, i: (0, j)),
            pl.BlockSpec((bi, f), lambda j, i: (i, 0)),
            pl.BlockSpec((1, 1), lambda j, i: (0, 0)),
        ],
        out_specs=pl.BlockSpec((bj, f), lambda j, i: (j, 0)),
        out_shape=jax.ShapeDtypeStruct((n, f), jnp.float32),
        scratch_shapes=[
            pltpu.VMEM((bj, f), jnp.float32),
            pltpu.VMEM((1, bj), jnp.float32),
        ],
        compiler_params=pltpu.CompilerParams(
            dimension_semantics=("parallel", "arbitrary")),
    )(adj, el, er, xv, elmax)
    return out


# speedup vs baseline: 1.5993x; 1.3202x over previous
import functools

import jax
import jax.numpy as jnp
from jax.experimental import pallas as pl
from jax.experimental.pallas import tpu as pltpu


def _proj_kernel(x_ref, wv_ref, bv_ref, wq_ref, bq_ref, wk_ref, bk_ref,
                 xv_ref, xvb_ref, el_ref, er_ref, elmax_ref):
    i = pl.program_id(0)
    xv = jax.lax.dot_general(
        x_ref[...], wv_ref[...], (((1,), (1,)), ((), ())),
        preferred_element_type=jnp.float32) + bv_ref[...]
    xv_ref[...] = xv
    xvb_ref[...] = xv.astype(jnp.bfloat16)
    el = jnp.sum(xv * wq_ref[...], axis=1, keepdims=True) + bq_ref[0, 0]
    el_ref[...] = el
    er_col = jnp.sum(xv * wk_ref[...], axis=1, keepdims=True) + bk_ref[0, 0]
    er_ref[...] = er_col.T
    bmax = jnp.max(el, keepdims=True)

    @pl.when(i == 0)
    def _():
        elmax_ref[...] = bmax

    @pl.when(i > 0)
    def _():
        elmax_ref[...] = jnp.maximum(elmax_ref[...], bmax)


def _attn_kernel(adj_ref, el_ref, er_ref, xvb_ref, elmax_ref, out_ref,
                 acc_ref, d_ref, *, ni):
    i = pl.program_id(1)

    @pl.when(i == 0)
    def _():
        acc_ref[...] = jnp.zeros_like(acc_ref)
        d_ref[...] = jnp.zeros_like(d_ref)

    el = el_ref[...]
    er = er_ref[...]
    e = el + er
    coeff = jnp.maximum(e, 0.2 * e)
    mtop = elmax_ref[...] + er
    mj = jnp.maximum(mtop, 0.2 * mtop)
    mask = adj_ref[...] > 0
    p = jnp.where(mask, jnp.exp(coeff - mj), 0.0)
    d_ref[...] += jnp.sum(p, axis=0, keepdims=True)
    acc_ref[...] += jax.lax.dot_general(
        p.astype(jnp.bfloat16), xvb_ref[...], (((0,), (0,)), ((), ())),
        preferred_element_type=jnp.float32)

    @pl.when(i == ni - 1)
    def _():
        d = jnp.maximum(d_ref[...], 1e-20)
        out_ref[...] = acc_ref[...] * (1.0 / d).T


def kernel(x, adj, Wv, bv, wq, bq, wk, bk):
    n, _ = x.shape
    f = Wv.shape[0]

    bi1 = min(512, n)
    ni1 = n // bi1
    xv, xvb, el, er, elmax = pl.pallas_call(
        _proj_kernel,
        grid=(ni1,),
        in_specs=[
            pl.BlockSpec((bi1, x.shape[1]), lambda i: (i, 0)),
            pl.BlockSpec(Wv.shape, lambda i: (0, 0)),
            pl.BlockSpec((1, f), lambda i: (0, 0)),
            pl.BlockSpec((1, f), lambda i: (0, 0)),
            pl.BlockSpec((1, 1), lambda i: (0, 0)),
            pl.BlockSpec((1, f), lambda i: (0, 0)),
            pl.BlockSpec((1, 1), lambda i: (0, 0)),
        ],
        out_specs=[
            pl.BlockSpec((bi1, f), lambda i: (i, 0)),
            pl.BlockSpec((bi1, f), lambda i: (i, 0)),
            pl.BlockSpec((bi1, 1), lambda i: (i, 0)),
            pl.BlockSpec((1, bi1), lambda i: (0, i)),
            pl.BlockSpec((1, 1), lambda i: (0, 0)),
        ],
        out_shape=[
            jax.ShapeDtypeStruct((n, f), jnp.float32),
            jax.ShapeDtypeStruct((n, f), jnp.bfloat16),
            jax.ShapeDtypeStruct((n, 1), jnp.float32),
            jax.ShapeDtypeStruct((1, n), jnp.float32),
            jax.ShapeDtypeStruct((1, 1), jnp.float32),
        ],
        compiler_params=pltpu.CompilerParams(
            dimension_semantics=("arbitrary",)),
    )(x, Wv, bv.reshape(1, f), wq, bq.reshape(1, 1), wk, bk.reshape(1, 1))
    del xv

    bi = min(1024, n)
    bj = min(512, n)
    ni = n // bi
    nj = n // bj
    out = pl.pallas_call(
        functools.partial(_attn_kernel, ni=ni),
        grid=(nj, ni),
        in_specs=[
            pl.BlockSpec((bi, bj), lambda j, i: (i, j)),
            pl.BlockSpec((bi, 1), lambda j, i: (i, 0)),
            pl.BlockSpec((1, bj), lambda j, i: (0, j)),
            pl.BlockSpec((bi, f), lambda j, i: (i, 0)),
            pl.BlockSpec((1, 1), lambda j, i: (0, 0)),
        ],
        out_specs=pl.BlockSpec((bj, f), lambda j, i: (j, 0)),
        out_shape=jax.ShapeDtypeStruct((n, f), jnp.float32),
        scratch_shapes=[
            pltpu.VMEM((bj, f), jnp.float32),
            pltpu.VMEM((1, bj), jnp.float32),
        ],
        compiler_params=pltpu.CompilerParams(
            dimension_semantics=("parallel", "arbitrary")),
    )(adj, el, er, xvb, elmax)
    return out


# speedup vs baseline: 1.8762x; 1.1732x over previous
import functools

import jax
import jax.numpy as jnp
from jax.experimental import pallas as pl
from jax.experimental.pallas import tpu as pltpu

_LOG2E = 1.4426950408889634


def _proj_kernel(x_ref, wv_ref, bv_ref, wq_ref, bq_ref, wk_ref, bk_ref,
                 xvb_ref, el_ref, er_ref, elmax_ref):
    i = pl.program_id(0)
    xv = jax.lax.dot_general(
        x_ref[...], wv_ref[...], (((1,), (1,)), ((), ())),
        preferred_element_type=jnp.float32) + bv_ref[...]
    xvb_ref[...] = xv.astype(jnp.bfloat16)
    el = jnp.sum(xv * wq_ref[...], axis=1, keepdims=True) + bq_ref[0, 0]
    el_ref[...] = el
    er_col = jnp.sum(xv * wk_ref[...], axis=1, keepdims=True) + bk_ref[0, 0]
    er_ref[...] = er_col.T
    bmax = jnp.max(el, keepdims=True)

    @pl.when(i == 0)
    def _():
        elmax_ref[...] = bmax

    @pl.when(i > 0)
    def _():
        elmax_ref[...] = jnp.maximum(elmax_ref[...], bmax)


def _attn_kernel(adj_ref, el_ref, er_ref, xvb_ref, elmax_ref, out_ref,
                 acc_ref, d_ref, *, ni, bi):
    i = pl.program_id(1)

    @pl.when(i == 0)
    def _():
        acc_ref[...] = jnp.zeros_like(acc_ref)
        d_ref[...] = jnp.zeros_like(d_ref)

    el = el_ref[...]
    er = er_ref[...]
    mtop = elmax_ref[...] + er
    mj = jnp.maximum(mtop, 0.2 * mtop)
    a = (er - mj) * _LOG2E
    b = (0.2 * er - mj) * _LOG2E
    elc = el * _LOG2E
    elc2 = elc * 0.2
    z = jnp.maximum(elc + a, elc2 + b)
    p = jnp.where(adj_ref[...] > 0, jnp.exp2(z), 0.0).astype(jnp.bfloat16)
    acc_ref[...] += jax.lax.dot_general(
        p, xvb_ref[...], (((0,), (0,)), ((), ())),
        preferred_element_type=jnp.float32)
    ones = jnp.ones((1, bi), dtype=jnp.bfloat16)
    d_ref[...] += jax.lax.dot_general(
        ones, p, (((1,), (0,)), ((), ())),
        preferred_element_type=jnp.float32)

    @pl.when(i == ni - 1)
    def _():
        d = jnp.maximum(d_ref[...], 1e-20)
        out_ref[...] = acc_ref[...] * (1.0 / d).T


def kernel(x, adj, Wv, bv, wq, bq, wk, bk):
    n, _ = x.shape
    f = Wv.shape[0]

    bi1 = min(512, n)
    ni1 = n // bi1
    xvb, el, er, elmax = pl.pallas_call(
        _proj_kernel,
        grid=(ni1,),
        in_specs=[
            pl.BlockSpec((bi1, x.shape[1]), lambda i: (i, 0)),
            pl.BlockSpec(Wv.shape, lambda i: (0, 0)),
            pl.BlockSpec((1, f), lambda i: (0, 0)),
            pl.BlockSpec((1, f), lambda i: (0, 0)),
            pl.BlockSpec((1, 1), lambda i: (0, 0)),
            pl.BlockSpec((1, f), lambda i: (0, 0)),
            pl.BlockSpec((1, 1), lambda i: (0, 0)),
        ],
        out_specs=[
            pl.BlockSpec((bi1, f), lambda i: (i, 0)),
            pl.BlockSpec((bi1, 1), lambda i: (i, 0)),
            pl.BlockSpec((1, bi1), lambda i: (0, i)),
            pl.BlockSpec((1, 1), lambda i: (0, 0)),
        ],
        out_shape=[
            jax.ShapeDtypeStruct((n, f), jnp.bfloat16),
            jax.ShapeDtypeStruct((n, 1), jnp.float32),
            jax.ShapeDtypeStruct((1, n), jnp.float32),
            jax.ShapeDtypeStruct((1, 1), jnp.float32),
        ],
        compiler_params=pltpu.CompilerParams(
            dimension_semantics=("arbitrary",)),
    )(x, Wv, bv.reshape(1, f), wq, bq.reshape(1, 1), wk, bk.reshape(1, 1))

    bi = min(2048, n)
    bj = min(512, n)
    ni = n // bi
    nj = n // bj
    out = pl.pallas_call(
        functools.partial(_attn_kernel, ni=ni, bi=bi),
        grid=(nj, ni),
        in_specs=[
            pl.BlockSpec((bi, bj), lambda j, i: (i, j)),
            pl.BlockSpec((bi, 1), lambda j, i: (i, 0)),
            pl.BlockSpec((1, bj), lambda j, i: (0, j)),
            pl.BlockSpec((bi, f), lambda j, i: (i, 0)),
            pl.BlockSpec((1, 1), lambda j, i: (0, 0)),
        ],
        out_specs=pl.BlockSpec((bj, f), lambda j, i: (j, 0)),
        out_shape=jax.ShapeDtypeStruct((n, f), jnp.float32),
        scratch_shapes=[
            pltpu.VMEM((bj, f), jnp.float32),
            pltpu.VMEM((1, bj), jnp.float32),
        ],
        compiler_params=pltpu.CompilerParams(
            dimension_semantics=("parallel", "arbitrary")),
    )(adj, el, er, xvb, elmax)
    return out
